# baseline (speedup 1.0000x reference)
import functools

import jax
import jax.numpy as jnp
from jax import lax
from jax.experimental import pallas as pl
from jax.experimental.pallas import tpu as pltpu
from jax.experimental.pallas import tpu_sc as plsc

N_ATOMS = 100000
N_MOL = 5000
NF = 128

BLK = 12288
GRID = 9
N_PAD = BLK * GRID
NS = 16
NWALL = 32
PER_W = N_PAD // NWALL
TAIL_W = N_ATOMS // PER_W
TAIL_START = TAIL_W * PER_W
TAIL = N_ATOMS - TAIL_START
ACC = N_MOL + 8


def _dense_body(f0, f1, f2, ch, w0, w1, w2, b1, b2, pe):
    t = (
        jnp.dot(f0[...], w0[...], preferred_element_type=jnp.float32)
        + jnp.dot(f1[...], w1[...], preferred_element_type=jnp.float32)
        + jnp.dot(f2[...], w2[...], preferred_element_type=jnp.float32)
        + b1[0, 0]
        + b2[0, 0]
    )
    tr = t.reshape(1, 1, BLK)
    pe[...] = (tr * ch[...]) ** 2 + tr


def _dense_call(f0, f1, f2, charges, w0, w1, w2, b1, b2):
    return pl.pallas_call(
        _dense_body,
        grid=(GRID,),
        in_specs=[
            pl.BlockSpec((BLK, NF), lambda i: (i, 0)),
            pl.BlockSpec((BLK, NF), lambda i: (i, 0)),
            pl.BlockSpec((BLK, NF), lambda i: (i, 0)),
            pl.BlockSpec((1, 1, BLK), lambda i: (i, 0, 0)),
            pl.BlockSpec((NF, 1), lambda i: (0, 0)),
            pl.BlockSpec((NF, 1), lambda i: (0, 0)),
            pl.BlockSpec((NF, 1), lambda i: (0, 0)),
            pl.BlockSpec((1, 1), lambda i: (0, 0)),
            pl.BlockSpec((1, 1), lambda i: (0, 0)),
        ],
        out_specs=pl.BlockSpec((1, 1, BLK), lambda i: (i, 0, 0)),
        out_shape=jax.ShapeDtypeStruct((GRID, 1, BLK), jnp.float32),
    )(f0, f1, f2, charges, w0, w1, w2, b1, b2)


@functools.partial(
    pl.kernel,
    mesh=plsc.VectorSubcoreMesh(core_axis_name="c", subcore_axis_name="s"),
    out_type=jax.ShapeDtypeStruct((2, ACC), jnp.float32),
    scratch_types=[
        pltpu.VMEM((PER_W,), jnp.int32),
        pltpu.VMEM((PER_W,), jnp.float32),
        pltpu.VMEM((TAIL,), jnp.int32),
        pltpu.VMEM((TAIL,), jnp.float32),
        pltpu.VMEM((ACC,), jnp.float32),
        pltpu.VMEM_SHARED((ACC,), jnp.float32),
    ],
)
def _sc_segsum(idx_hbm, en_hbm, out_hbm, idx_v, en_v, idx_t, en_t, zbuf, accum):
    cid = lax.axis_index("c")
    sid = lax.axis_index("s")
    wid = cid * NS + sid

    @pl.when(sid == 0)
    def _():
        def zb(i, c):
            zbuf[pl.ds(i * 16, 16)] = jnp.zeros((16,), jnp.float32)
            return c

        lax.fori_loop(0, ACC // 16, zb, 0)
        pltpu.sync_copy(zbuf, accum)

    plsc.subcore_barrier()

    @pl.when(wid < TAIL_W)
    def _():
        start = wid * PER_W
        pltpu.sync_copy(idx_hbm.at[pl.ds(start, PER_W)], idx_v)
        pltpu.sync_copy(en_hbm.at[pl.ds(start, PER_W)], en_v)
        pltpu.sync_copy(en_v, accum.at[idx_v], add=True)

    @pl.when(wid == TAIL_W)
    def _():
        pltpu.sync_copy(idx_hbm.at[pl.ds(TAIL_START, TAIL)], idx_t)
        pltpu.sync_copy(en_hbm.at[pl.ds(TAIL_START, TAIL)], en_t)
        pltpu.sync_copy(en_t, accum.at[idx_t], add=True)

    plsc.subcore_barrier()

    @pl.when(sid == 0)
    def _():
        pltpu.sync_copy(accum, out_hbm.at[cid])


def kernel(charges, f0, f1, f2, Wl0, Wl1, Wl2, bl1, bl2, mol_index, n_molecules):
    chp = jnp.concatenate(
        [charges.reshape(N_ATOMS),
         jnp.zeros((N_PAD - N_ATOMS,), jnp.float32)]
    ).reshape(GRID, 1, BLK)
    pe = _dense_call(
        f0, f1, f2, chp, Wl0, Wl1, Wl2,
        bl1.reshape(1, 1), bl2.reshape(1, 1),
    )
    flat = pe.reshape(N_PAD)
    ae = flat[:N_ATOMS].reshape(N_ATOMS, 1)
    acc = _sc_segsum(mol_index.astype(jnp.int32), flat)
    mol = (acc[0, :N_MOL] + acc[1, :N_MOL]).reshape(N_MOL, 1)
    return (mol, ae)


# speedup vs baseline: 1.0390x; 1.0390x over previous
import functools

import jax
import jax.numpy as jnp
from jax import lax
from jax.experimental import pallas as pl
from jax.experimental.pallas import tpu as pltpu
from jax.experimental.pallas import tpu_sc as plsc

N_ATOMS = 100000
N_MOL = 5000
NF = 128

BLK = 8192
GRID = 13
N_PAD = BLK * GRID
NS = 16
NWALL = 32
PER_W = N_PAD // NWALL
TAIL_W = N_ATOMS // PER_W
TAIL_START = TAIL_W * PER_W
TAIL = N_ATOMS - TAIL_START
ACC = N_MOL + 8


def _dense_body(f0, f1, f2, ch, w0, w1, w2, b1, b2, pe):
    t = (
        jnp.dot(f0[...], w0[...], preferred_element_type=jnp.float32)
        + jnp.dot(f1[...], w1[...], preferred_element_type=jnp.float32)
        + jnp.dot(f2[...], w2[...], preferred_element_type=jnp.float32)
        + b1[0, 0]
        + b2[0, 0]
    )
    tr = t.reshape(1, 1, BLK)
    pe[...] = (tr * ch[...]) ** 2 + tr


def _dense_call(f0, f1, f2, charges, w0, w1, w2, b1, b2):
    return pl.pallas_call(
        _dense_body,
        grid=(GRID,),
        in_specs=[
            pl.BlockSpec((BLK, NF), lambda i: (i, 0)),
            pl.BlockSpec((BLK, NF), lambda i: (i, 0)),
            pl.BlockSpec((BLK, NF), lambda i: (i, 0)),
            pl.BlockSpec((1, 1, BLK), lambda i: (i, 0, 0)),
            pl.BlockSpec((NF, 1), lambda i: (0, 0)),
            pl.BlockSpec((NF, 1), lambda i: (0, 0)),
            pl.BlockSpec((NF, 1), lambda i: (0, 0)),
            pl.BlockSpec((1, 1), lambda i: (0, 0)),
            pl.BlockSpec((1, 1), lambda i: (0, 0)),
        ],
        out_specs=pl.BlockSpec((1, 1, BLK), lambda i: (i, 0, 0)),
        out_shape=jax.ShapeDtypeStruct((GRID, 1, BLK), jnp.float32),
    )(f0, f1, f2, charges, w0, w1, w2, b1, b2)


@functools.partial(
    pl.kernel,
    mesh=plsc.VectorSubcoreMesh(core_axis_name="c", subcore_axis_name="s"),
    out_type=jax.ShapeDtypeStruct((2, ACC), jnp.float32),
    scratch_types=[
        pltpu.VMEM((PER_W,), jnp.int32),
        pltpu.VMEM((PER_W,), jnp.float32),
        pltpu.VMEM((TAIL,), jnp.int32),
        pltpu.VMEM((TAIL,), jnp.float32),
        pltpu.VMEM((ACC,), jnp.float32),
        pltpu.VMEM_SHARED((ACC,), jnp.float32),
    ],
)
def _sc_segsum(idx_hbm, en_hbm, out_hbm, idx_v, en_v, idx_t, en_t, zbuf, accum):
    cid = lax.axis_index("c")
    sid = lax.axis_index("s")
    wid = cid * NS + sid

    @pl.when(sid == 0)
    def _():
        def zb(i, c):
            zbuf[pl.ds(i * 16, 16)] = jnp.zeros((16,), jnp.float32)
            return c

        lax.fori_loop(0, ACC // 16, zb, 0)
        pltpu.sync_copy(zbuf, accum)

    plsc.subcore_barrier()

    @pl.when(wid < TAIL_W)
    def _():
        start = wid * PER_W
        pltpu.sync_copy(idx_hbm.at[pl.ds(start, PER_W)], idx_v)
        pltpu.sync_copy(en_hbm.at[pl.ds(start, PER_W)], en_v)
        pltpu.sync_copy(en_v, accum.at[idx_v], add=True)

    @pl.when(wid == TAIL_W)
    def _():
        pltpu.sync_copy(idx_hbm.at[pl.ds(TAIL_START, TAIL)], idx_t)
        pltpu.sync_copy(en_hbm.at[pl.ds(TAIL_START, TAIL)], en_t)
        pltpu.sync_copy(en_t, accum.at[idx_t], add=True)

    plsc.subcore_barrier()

    @pl.when(sid == 0)
    def _():
        pltpu.sync_copy(accum, out_hbm.at[cid])


def kernel(charges, f0, f1, f2, Wl0, Wl1, Wl2, bl1, bl2, mol_index, n_molecules):
    chp = jnp.concatenate(
        [charges.reshape(N_ATOMS),
         jnp.zeros((N_PAD - N_ATOMS,), jnp.float32)]
    ).reshape(GRID, 1, BLK)
    pe = _dense_call(
        f0, f1, f2, chp, Wl0, Wl1, Wl2,
        bl1.reshape(1, 1), bl2.reshape(1, 1),
    )
    flat = pe.reshape(N_PAD)
    ae = flat[:N_ATOMS].reshape(N_ATOMS, 1)
    acc = _sc_segsum(mol_index.astype(jnp.int32), flat)
    mol = (acc[0, :N_MOL] + acc[1, :N_MOL]).reshape(N_MOL, 1)
    return (mol, ae)


# speedup vs baseline: 1.0514x; 1.0119x over previous
import functools

import jax
import jax.numpy as jnp
from jax import lax
from jax.experimental import pallas as pl
from jax.experimental.pallas import tpu as pltpu
from jax.experimental.pallas import tpu_sc as plsc

N_ATOMS = 100000
N_MOL = 5000
NF = 128

BLK = 4096
GRID = 25
N_PAD = BLK * GRID
NS = 16
NWALL = 32
PER_W = N_PAD // NWALL
TAIL_W = N_ATOMS // PER_W
TAIL_START = TAIL_W * PER_W
TAIL = N_ATOMS - TAIL_START
ACC = N_MOL + 8


def _dense_body(f0, f1, f2, ch, w0, w1, w2, b1, b2, pe):
    t = (
        jnp.dot(f0[...], w0[...], preferred_element_type=jnp.float32)
        + jnp.dot(f1[...], w1[...], preferred_element_type=jnp.float32)
        + jnp.dot(f2[...], w2[...], preferred_element_type=jnp.float32)
        + b1[0, 0]
        + b2[0, 0]
    )
    tr = t.reshape(1, 1, BLK)
    pe[...] = (tr * ch[...]) ** 2 + tr


def _dense_call(f0, f1, f2, charges, w0, w1, w2, b1, b2):
    return pl.pallas_call(
        _dense_body,
        grid=(GRID,),
        in_specs=[
            pl.BlockSpec((BLK, NF), lambda i: (i, 0)),
            pl.BlockSpec((BLK, NF), lambda i: (i, 0)),
            pl.BlockSpec((BLK, NF), lambda i: (i, 0)),
            pl.BlockSpec((1, 1, BLK), lambda i: (i, 0, 0)),
            pl.BlockSpec((NF, 1), lambda i: (0, 0)),
            pl.BlockSpec((NF, 1), lambda i: (0, 0)),
            pl.BlockSpec((NF, 1), lambda i: (0, 0)),
            pl.BlockSpec((1, 1), lambda i: (0, 0)),
            pl.BlockSpec((1, 1), lambda i: (0, 0)),
        ],
        out_specs=pl.BlockSpec((1, 1, BLK), lambda i: (i, 0, 0)),
        out_shape=jax.ShapeDtypeStruct((GRID, 1, BLK), jnp.float32),
    )(f0, f1, f2, charges, w0, w1, w2, b1, b2)


@functools.partial(
    pl.kernel,
    mesh=plsc.VectorSubcoreMesh(core_axis_name="c", subcore_axis_name="s"),
    out_type=jax.ShapeDtypeStruct((2, ACC), jnp.float32),
    scratch_types=[
        pltpu.VMEM((PER_W,), jnp.int32),
        pltpu.VMEM((PER_W,), jnp.float32),
        pltpu.VMEM((TAIL,), jnp.int32),
        pltpu.VMEM((TAIL,), jnp.float32),
        pltpu.VMEM((ACC,), jnp.float32),
        pltpu.VMEM_SHARED((ACC,), jnp.float32),
    ],
)
def _sc_segsum(idx_hbm, en_hbm, out_hbm, idx_v, en_v, idx_t, en_t, zbuf, accum):
    cid = lax.axis_index("c")
    sid = lax.axis_index("s")
    wid = cid * NS + sid

    @pl.when(sid == 0)
    def _():
        def zb(i, c):
            zbuf[pl.ds(i * 16, 16)] = jnp.zeros((16,), jnp.float32)
            return c

        lax.fori_loop(0, ACC // 16, zb, 0)
        pltpu.sync_copy(zbuf, accum)

    plsc.subcore_barrier()

    @pl.when(wid < TAIL_W)
    def _():
        start = wid * PER_W
        pltpu.sync_copy(idx_hbm.at[pl.ds(start, PER_W)], idx_v)
        pltpu.sync_copy(en_hbm.at[pl.ds(start, PER_W)], en_v)
        pltpu.sync_copy(en_v, accum.at[idx_v], add=True)

    @pl.when(wid == TAIL_W)
    def _():
        pltpu.sync_copy(idx_hbm.at[pl.ds(TAIL_START, TAIL)], idx_t)
        pltpu.sync_copy(en_hbm.at[pl.ds(TAIL_START, TAIL)], en_t)
        pltpu.sync_copy(en_t, accum.at[idx_t], add=True)

    plsc.subcore_barrier()

    @pl.when(sid == 0)
    def _():
        pltpu.sync_copy(accum, out_hbm.at[cid])


def kernel(charges, f0, f1, f2, Wl0, Wl1, Wl2, bl1, bl2, mol_index, n_molecules):
    chp = jnp.concatenate(
        [charges.reshape(N_ATOMS),
         jnp.zeros((N_PAD - N_ATOMS,), jnp.float32)]
    ).reshape(GRID, 1, BLK)
    pe = _dense_call(
        f0, f1, f2, chp, Wl0, Wl1, Wl2,
        bl1.reshape(1, 1), bl2.reshape(1, 1),
    )
    flat = pe.reshape(N_PAD)
    ae = flat[:N_ATOMS].reshape(N_ATOMS, 1)
    acc = _sc_segsum(mol_index.astype(jnp.int32), flat)
    mol = (acc[0, :N_MOL] + acc[1, :N_MOL]).reshape(N_MOL, 1)
    return (mol, ae)
